# baseline (device time: 40962 ns/iter reference)
import jax
import jax.numpy as jnp
from jax import lax
from jax.experimental import pallas as pl
from jax.experimental.pallas import tpu as pltpu

N_ROWS = 2048
C = 128
PAD = N_ROWS + C
MAX_CHUNKS = 17


def kernel(x, dest):
    n, d = x.shape
    me = lax.axis_index("x")

    is0 = (dest == 0).astype(jnp.int32)
    c0 = jnp.cumsum(is0)
    total0 = c0[-1]
    i = jnp.arange(n, dtype=jnp.int32)
    q = i + 1
    ss0 = jnp.searchsorted(c0, q, side="left", method="compare_all")
    ss1 = jnp.searchsorted(q - c0, q, side="left", method="compare_all")

    kp = jnp.where(me == 0, total0, n - total0)
    k = n - kp
    s0 = kp % 8
    S0 = jnp.where(s0 > 0, kp - s0 + 8, kp)
    K0 = jnp.where(s0 > 0, k + 8, k)

    zpad = jnp.zeros((PAD - n,), jnp.int32)
    ss0p = jnp.concatenate([ss0, zpad])
    ss1p = jnp.concatenate([ss1, zpad])
    a = jnp.roll(ss0p, jnp.where(me == 0, 0, s0))
    b = jnp.roll(ss1p, jnp.where(me == 0, S0, K0))
    t = jnp.where(me == 0, kp, K0)
    perm = jnp.where(jnp.arange(PAD, dtype=jnp.int32) < t, a, b)
    xs = x[perm].astype(jnp.bfloat16)

    def body(kp_ref, x_ref, out_ref, send_sems, recv_sems, local_sems):
        my_x = lax.axis_index("x")
        my_y = lax.axis_index("y")
        partner = (1 - my_x, my_y)
        im0 = my_x == 0

        kp = kp_ref[0]
        k = n - kp
        s0 = lax.rem(kp, 8)
        s1 = lax.rem(k, 8)
        kp_down = kp - s0
        k_down = k - s1
        k_up = jnp.where(s1 > 0, k_down + 8, k)
        S0 = jnp.where(s0 > 0, kp_down + 8, kp)
        K0 = jnp.where(s0 > 0, k + 8, k)
        D = K0 - k
        F01 = (k_down + C - 1) // C
        F10 = (n - kp_down + C - 1) // C
        L0 = (kp_down + C - 1) // C
        L1 = (n - k_up + C - 1) // C

        def al(v):
            return pl.multiple_of(v, 8)

        def remote(src, dst, j, rows=C):
            return pltpu.make_async_remote_copy(
                src_ref=x_ref.at[pl.ds(al(src), rows), :],
                dst_ref=out_ref.at[pl.ds(al(dst), rows), :],
                send_sem=send_sems.at[j],
                recv_sem=recv_sems.at[j],
                device_id=partner,
                device_id_type=pl.DeviceIdType.MESH,
            )

        def local(src, dst, j):
            return pltpu.make_async_copy(
                x_ref.at[pl.ds(al(src), C), :],
                out_ref.at[pl.ds(al(dst), C), :],
                local_sems.at[j],
            )

        barrier = pltpu.get_barrier_semaphore()
        pl.semaphore_signal(
            barrier, inc=1, device_id=partner,
            device_id_type=pl.DeviceIdType.MESH,
        )

        def local0(j):
            off = jnp.where(j == L0 - 1, kp_down - C, j * C)
            return local(off, off, j)

        def local1(j):
            dst = jnp.where(j == L1 - 1, n - C, k_up + j * C)
            return local(dst + D, dst, j)

        for j in range(MAX_CHUNKS):
            @pl.when(im0 & (j < L0))
            def _():
                local0(j).start()

            @pl.when((~im0) & (j < L1))
            def _():
                local1(j).start()

        pl.semaphore_wait(barrier, 1)

        def send01(j):
            src = jnp.where(j == F01 - 1, S0 + k_down - C, S0 + j * C)
            dst = jnp.where(j == F01 - 1, k_down - C, j * C)
            return remote(src, dst, j)

        def send10(j):
            src = jnp.where(j == F10 - 1, n - C - kp_down, j * C)
            dst = jnp.where(j == F10 - 1, n - C, kp_down + j * C)
            return remote(src, dst, j)

        for j in range(MAX_CHUNKS):
            @pl.when(im0 & (j < F01))
            def _():
                send01(j).start()

            @pl.when(im0 & (j == F01) & (s1 > 0))
            def _():
                remote(S0 + k_down, k_down, j, rows=8).start()

            @pl.when((~im0) & (j < F10))
            def _():
                send10(j).start()

        for j in range(MAX_CHUNKS):
            @pl.when(im0 & (j < L0))
            def _():
                local0(j).wait()

            @pl.when((~im0) & (j < L1))
            def _():
                local1(j).wait()

        for j in range(MAX_CHUNKS):
            @pl.when(im0 & (j < F01))
            def _():
                send01(j).wait_send()

            @pl.when(im0 & (j == F01) & (s1 > 0))
            def _():
                remote(S0 + k_down, k_down, j, rows=8).wait_send()

            @pl.when((~im0) & (j < F10))
            def _():
                send10(j).wait_send()

        for j in range(MAX_CHUNKS):
            @pl.when(im0 & (j < F10))
            def _():
                send10(j).wait_recv()

            @pl.when((~im0) & (j < F01))
            def _():
                send01(j).wait_recv()

            @pl.when((~im0) & (j == F01) & (s1 > 0))
            def _():
                remote(S0 + k_down, k_down, j, rows=8).wait_recv()

        u = lax.broadcasted_iota(jnp.int32, (8, 1), 0)

        @pl.when(im0 & (s0 > 0))
        def _():
            w = out_ref[pl.ds(al(kp_down), 8), :]
            m = x_ref[pl.ds(al(kp_down), 8), :]
            out_ref[pl.ds(al(kp_down), 8), :] = jnp.where(u < s0, m, w)

        @pl.when((~im0) & (s1 > 0))
        def _():
            w = out_ref[pl.ds(al(k_down), 8), :]
            m = x_ref[pl.ds(al(K0 - s1), 8), :]
            out_ref[pl.ds(al(k_down), 8), :] = jnp.where(u < s1, w, m)

    return pl.pallas_call(
        body,
        out_shape=jax.ShapeDtypeStruct((n, d), jnp.bfloat16),
        in_specs=[
            pl.BlockSpec(memory_space=pltpu.SMEM),
            pl.BlockSpec(memory_space=pltpu.VMEM),
        ],
        out_specs=pl.BlockSpec(memory_space=pltpu.VMEM),
        scratch_shapes=[
            pltpu.SemaphoreType.DMA((MAX_CHUNKS,)),
            pltpu.SemaphoreType.DMA((MAX_CHUNKS,)),
            pltpu.SemaphoreType.DMA((MAX_CHUNKS,)),
        ],
        compiler_params=pltpu.CompilerParams(collective_id=0),
    )(kp.reshape(1), xs)
